# baseline (speedup 1.0000x reference)
import functools

import jax
import jax.numpy as jnp
from jax import lax
from jax.experimental import pallas as pl
from jax.experimental.pallas import tpu as pltpu
from jax.experimental.pallas import tpu_sc as plsc

N = 10000
D = 128
H = D // 2
E = 160000
NC = 2
NS = 16
NW = NC * NS
C = 128
RPW = 40
E_PAD = NW * RPW * C
ROWS = E_PAD // C
REAL_ROWS = E // C
NPT = 624
ZR = 104
REM = N - NS * NPT

_MESH = plsc.VectorSubcoreMesh(
    core_axis_name="c", subcore_axis_name="s", num_cores=NC, num_subcores=NS)

_SC_PARAMS = pltpu.CompilerParams(use_tc_tiling_on_sc=False)

_GDN = lax.GatherDimensionNumbers(
    offset_dims=(), collapsed_slice_dims=(0,), start_index_map=(0,))


def _tree_sum(ps):
    while len(ps) > 1:
        ps = [a + b for a, b in zip(ps[::2], ps[1::2])]
    return ps[0]


def _shuf(v, idx):
    return lax.gather(v, idx.reshape(16, 1), _GDN, (1,),
                      indices_are_sorted=False, unique_indices=False,
                      mode=lax.GatherScatterMode.PROMISE_IN_BOUNDS)


def _lane_bcast(v, i):
    return _shuf(v, jnp.full((16,), i, jnp.int32))


def _transpose_reduce(accs):
    lanes = lax.iota(jnp.int32, 16)
    for sh in (1, 2, 4, 8):
        bit = (lanes & sh) == 0
        xidx = lanes ^ sh
        nxt = []
        for k in range(0, len(accs), 2):
            a, b = accs[k], accs[k + 1]
            m1 = jnp.where(bit, a, b)
            m2 = jnp.where(bit, b, a)
            nxt.append(m1 + _shuf(m2, xidx))
        accs = nxt
    return accs[0]


def _worker_prelude(src_h, dst_h, m_h, idxs_v, idxd_v, m_v):
    c = lax.axis_index("c")
    s = lax.axis_index("s")
    w = s * NC + c
    r0 = w * RPW
    pltpu.sync_copy(src_h.at[pl.ds(r0, RPW)], idxs_v)
    pltpu.sync_copy(dst_h.at[pl.ds(r0, RPW)], idxd_v)
    pltpu.sync_copy(m_h.at[pl.ds(r0, RPW)], m_v)
    return c, s, r0


def _affine_idx(out_v, src_v, mul, add):
    def row(j, _):
        for q in range(C // 16):
            sl = pl.ds(16 * q, 16)
            out_v[j, sl] = src_v[j, sl] * mul + add
        return 0
    lax.fori_loop(0, RPW, row, 0)


def _zero_acc(rows_v, acc_sh, s):
    def zrow(i, _):
        for k in range(H // 16):
            rows_v[i, pl.ds(16 * k, 16)] = jnp.zeros((16,), jnp.float32)
        return 0
    lax.fori_loop(0, ZR, zrow, 0)
    for k in range(NPT // ZR):
        base = pl.multiple_of(s * NPT + k * ZR, 8)
        pltpu.sync_copy(rows_v.at[pl.ds(0, ZR)], acc_sh.at[pl.ds(base, ZR)])

    @pl.when(s == NS - 1)
    def _():
        pltpu.sync_copy(rows_v.at[pl.ds(0, REM)],
                        acc_sh.at[pl.ds(NS * NPT, REM)])
    plsc.subcore_barrier()


def _acc_to_hbm(acc_sh, out_slice, rows_v, s):
    plsc.subcore_barrier()
    for k in range(NPT // ZR):
        base = pl.multiple_of(s * NPT + k * ZR, 8)
        pltpu.sync_copy(acc_sh.at[pl.ds(base, ZR)], rows_v.at[pl.ds(0, ZR)])
        pltpu.sync_copy(rows_v.at[pl.ds(0, ZR)], out_slice.at[pl.ds(base, ZR)])

    @pl.when(s == NS - 1)
    def _():
        pltpu.sync_copy(acc_sh.at[pl.ds(NS * NPT, REM)],
                        rows_v.at[pl.ds(0, REM)])
        pltpu.sync_copy(rows_v.at[pl.ds(0, REM)],
                        out_slice.at[pl.ds(NS * NPT, REM)])
    plsc.subcore_barrier()


NBUF = 4
NBUF1 = 2


@functools.partial(
    pl.kernel, mesh=_MESH, compiler_params=_SC_PARAMS,
    out_type=jax.ShapeDtypeStruct((NC, 2, N, H), jnp.float32),
    scratch_types=[
        pltpu.VMEM((RPW, C), jnp.int32),
        pltpu.VMEM((RPW, C), jnp.int32),
        pltpu.VMEM((RPW, C), jnp.float32),
        [pltpu.VMEM((C, H), jnp.float32)] * NBUF,
        pltpu.VMEM_SHARED((N, H), jnp.float32),
        [pltpu.SemaphoreType.DMA] * NBUF,
        [pltpu.SemaphoreType.DMA] * NBUF,
    ],
)
def _sc_spmm(tab2_h, src_h, dst_h, m_h, out_h,
             idxs_v, idxd_v, m_v, rows_bufs, acc_sh, gsems, ssems):
    c, s, _ = _worker_prelude(src_h, dst_h, m_h, idxs_v, idxd_v, m_v)
    for h in range(2):
        _affine_idx(idxs_v, idxs_v, 2 if h == 0 else 1, h)
        _zero_acc(rows_bufs[0], acc_sh, s)

        def quad(jj, _):
            j0 = jj * NBUF
            gets = [pltpu.async_copy(tab2_h.at[idxs_v.at[j0 + k]],
                                     rows_bufs[k], gsems[k])
                    for k in range(NBUF)]
            puts = []
            for k in range(NBUF):
                gets[k].wait()
                j = j0 + k
                rows_v = rows_bufs[k]

                @plsc.parallel_loop(0, C // 16, unroll=4)
                def group(g, j=j, rows_v=rows_v):
                    mv = m_v[j, pl.ds(16 * g, 16)]
                    for i in range(16):
                        e = 16 * g + i
                        mvec = _lane_bcast(mv, i)
                        for q in range(H // 16):
                            sl = pl.ds(16 * q, 16)
                            rows_v[e, sl] = rows_v[e, sl] * mvec
                puts.append(pltpu.async_copy(
                    rows_v, acc_sh.at[idxd_v.at[j]], ssems[k], add=True))
            for p in puts:
                p.wait()
            return 0
        lax.fori_loop(0, RPW // NBUF, quad, 0)
        _acc_to_hbm(acc_sh, out_h.at[c, h], rows_bufs[0], s)


@functools.partial(
    pl.kernel, mesh=_MESH, compiler_params=_SC_PARAMS,
    out_type=(jax.ShapeDtypeStruct((ROWS, C), jnp.float32),
              jax.ShapeDtypeStruct((NC, 2, N, H), jnp.float32)),
    scratch_types=[
        pltpu.VMEM((RPW, C), jnp.int32),
        pltpu.VMEM((RPW, C), jnp.int32),
        pltpu.VMEM((RPW, C), jnp.float32),
        pltpu.VMEM((RPW, C), jnp.float32),
        pltpu.VMEM((RPW, C), jnp.int32),
        [pltpu.VMEM((C, H), jnp.float32)] * NBUF1,
        [pltpu.VMEM((C, H), jnp.float32)] * NBUF1,
        pltpu.VMEM_SHARED((N, H), jnp.float32),
        [pltpu.SemaphoreType.DMA] * NBUF1,
        [pltpu.SemaphoreType.DMA] * NBUF1,
        [pltpu.SemaphoreType.DMA] * NBUF1,
    ],
)
def _sc_bwd1(dz2_h, h1w2_h, src_h, dst_h, m_h, ge2_h, out_h,
             idxs_v, ga_v, m_v, ge2_v, gb_v, abufs, bbufs,
             acc_sh, asems, bsems, ssems):
    c, s, r0 = _worker_prelude(src_h, dst_h, m_h, idxs_v, ga_v, m_v)
    for h in range(2):
        _affine_idx(ga_v, ga_v, 2 if h == 0 else 1, h)
        if h == 0:
            _affine_idx(gb_v, idxs_v, 2, 0)
        else:
            _affine_idx(gb_v, gb_v, 1, 1)
        _zero_acc(abufs[0], acc_sh, s)

        def quad(jj, _):
            j0 = jj * NBUF1
            gets = [(pltpu.async_copy(dz2_h.at[ga_v.at[j0 + k]],
                                      abufs[k], asems[k]),
                     pltpu.async_copy(h1w2_h.at[gb_v.at[j0 + k]],
                                      bbufs[k], bsems[k]))
                    for k in range(NBUF1)]
            puts = []
            for k in range(NBUF1):
                gets[k][0].wait()
                gets[k][1].wait()
                j = j0 + k
                rowsa_v, rowsb_v = abufs[k], bbufs[k]

                @plsc.parallel_loop(0, C // 16, unroll=4)
                def group(g, j=j, rowsa_v=rowsa_v, rowsb_v=rowsb_v):
                    sl16 = pl.ds(16 * g, 16)
                    mv = m_v[j, sl16]
                    accs = []
                    for i in range(16):
                        e = 16 * g + i
                        mvec = _lane_bcast(mv, i)
                        ps = []
                        for q in range(H // 16):
                            sl = pl.ds(16 * q, 16)
                            a = rowsa_v[e, sl]
                            ps.append(a * rowsb_v[e, sl])
                            rowsa_v[e, sl] = a * mvec
                        accs.append(_tree_sum(ps))
                    gvec = _transpose_reduce(accs)
                    if h == 0:
                        ge2_v[j, sl16] = gvec
                    else:
                        ge2_v[j, sl16] = ge2_v[j, sl16] + gvec
                puts.append(pltpu.async_copy(
                    rowsa_v, acc_sh.at[idxs_v.at[j]], ssems[k], add=True))
            for p in puts:
                p.wait()
            return 0
        lax.fori_loop(0, RPW // NBUF1, quad, 0)
        _acc_to_hbm(acc_sh, out_h.at[c, h], abufs[0], s)
    pltpu.sync_copy(ge2_v, ge2_h.at[pl.ds(r0, RPW)])


@functools.partial(
    pl.kernel, mesh=_MESH, compiler_params=_SC_PARAMS,
    out_type=jax.ShapeDtypeStruct((ROWS, C), jnp.float32),
    scratch_types=[
        pltpu.VMEM((RPW, C), jnp.int32),
        pltpu.VMEM((RPW, C), jnp.int32),
        pltpu.VMEM((RPW, C), jnp.float32),
        pltpu.VMEM((RPW, C), jnp.float32),
        pltpu.VMEM((16,), jnp.float32),
        [pltpu.VMEM((C, D), jnp.float32)] * 2,
        [pltpu.VMEM((C, D), jnp.float32)] * 2,
        [pltpu.SemaphoreType.DMA] * 2,
        [pltpu.SemaphoreType.DMA] * 2,
    ],
)
def _sc_bwd2(dz1_h, xw1_h, src_h, dst_h, m_h, ge2_h, corr_h, mout_h,
             idxs_v, idxd_v, m_v, ge2_v, corr_v, abufs, bbufs,
             asems, bsems):
    _, _, r0 = _worker_prelude(src_h, dst_h, m_h, idxs_v, idxd_v, m_v)
    pltpu.sync_copy(ge2_h.at[pl.ds(r0, RPW)], ge2_v)
    pltpu.sync_copy(corr_h, corr_v)
    corr = corr_v[...]

    def pair(jj, _):
        j0 = jj * 2
        gets = [(pltpu.async_copy(dz1_h.at[idxd_v.at[j0 + k]],
                                  abufs[k], asems[k]),
                 pltpu.async_copy(xw1_h.at[idxs_v.at[j0 + k]],
                                  bbufs[k], bsems[k]))
                for k in range(2)]
        for k in range(2):
            gets[k][0].wait()
            gets[k][1].wait()
            j = j0 + k
            rowsa_v, rowsb_v = abufs[k], bbufs[k]

            gj = r0 + j
            live = jnp.where(gj < REAL_ROWS, 1.0, 0.0).astype(jnp.float32)

            @plsc.parallel_loop(0, C // 16, unroll=4)
            def group(g, j=j, rowsa_v=rowsa_v, rowsb_v=rowsb_v, live=live):
                sl16 = pl.ds(16 * g, 16)
                accs = []
                for i in range(16):
                    e = 16 * g + i
                    ps = [rowsa_v[e, pl.ds(16 * q, 16)]
                          * rowsb_v[e, pl.ds(16 * q, 16)]
                          for q in range(D // 16)]
                    accs.append(_tree_sum(ps))
                gtot = _transpose_reduce(accs) + ge2_v[j, sl16] + corr
                m_v[j, sl16] = jnp.clip(m_v[j, sl16] + 0.005 * gtot,
                                        0.0, 1.0) * live
        return 0
    lax.fori_loop(0, RPW // 2, pair, 0)
    pltpu.sync_copy(m_v, mout_h.at[pl.ds(r0, RPW)])


def _tc_pre_body(x_ref, w1_ref, o_ref):
    o_ref[...] = jnp.dot(x_ref[...], w1_ref[...],
                         preferred_element_type=jnp.float32)


def _assemble(p_ref, b_ref):
    z = jnp.concatenate([p_ref[0, 0] + p_ref[1, 0],
                         p_ref[0, 1] + p_ref[1, 1]], axis=-1)
    return z + b_ref[...]


def _tc_fwd1_body(z1p_ref, b1_ref, w2_ref, z1_ref, h1w2_ref):
    z1 = _assemble(z1p_ref, b1_ref)
    z1_ref[...] = z1
    h1 = jnp.maximum(z1, 0.0)
    h1w2_ref[...] = jnp.dot(h1, w2_ref[...],
                            preferred_element_type=jnp.float32)


def _tc_dz2_body(z2p_ref, b2_ref, v2_ref, m_ref, t_ref, dz2_ref, corr_ref):
    z2 = _assemble(z2p_ref, b2_ref)
    dz2_ref[...] = jnp.where(z2 > 0.0, v2_ref[...], 0.0)
    corr = 2.0 * (t_ref[0, 0] - jnp.sum(m_ref[...]))
    corr_ref[...] = jnp.full((1, 128), corr, jnp.float32)


def _tc_dz1_body(sp_ref, w2_ref, z1_ref, dz1_ref):
    srow = jnp.concatenate([sp_ref[0, 0] + sp_ref[1, 0],
                            sp_ref[0, 1] + sp_ref[1, 1]], axis=-1)
    dh1 = lax.dot_general(srow, w2_ref[...], (((1,), (1,)), ((), ())),
                          preferred_element_type=jnp.float32)
    dz1_ref[...] = jnp.where(z1_ref[...] > 0.0, dh1, 0.0)


def kernel(x, edge_index, nodes, target_label, steps, W1, b1, W2, b2, Wlin, blin):
    n = x.shape[0]
    src = edge_index[0].astype(jnp.int32)
    dst = edge_index[1].astype(jnp.int32)
    pad = E_PAD - E
    src2d = jnp.concatenate([src, jnp.zeros((pad,), jnp.int32)]).reshape(ROWS, C)
    dst2d = jnp.concatenate([dst, jnp.zeros((pad,), jnp.int32)]).reshape(ROWS, C)
    m2d = jnp.concatenate([jnp.full((E,), 0.5, jnp.float32),
                           jnp.zeros((pad,), jnp.float32)]).reshape(ROWS, C)
    v2 = (Wlin[:, target_label] / nodes).astype(jnp.float32).reshape(1, D)
    t_arr = jnp.where(target_label == 0, 9.0, 8.0).astype(jnp.float32).reshape(1, 1)
    b1r = b1.reshape(1, D)
    b2r = b2.reshape(1, D)

    xW1 = pl.pallas_call(
        _tc_pre_body,
        out_shape=jax.ShapeDtypeStruct((n, D), jnp.float32),
    )(x, W1)
    xW1_2 = xW1.reshape(2 * n, H)

    tc_fwd1 = pl.pallas_call(
        _tc_fwd1_body,
        out_shape=(jax.ShapeDtypeStruct((n, D), jnp.float32),
                   jax.ShapeDtypeStruct((n, D), jnp.float32)),
    )
    tc_dz2 = pl.pallas_call(
        _tc_dz2_body,
        out_shape=(jax.ShapeDtypeStruct((n, D), jnp.float32),
                   jax.ShapeDtypeStruct((1, 128), jnp.float32)),
    )
    tc_dz1 = pl.pallas_call(
        _tc_dz1_body,
        out_shape=jax.ShapeDtypeStruct((n, D), jnp.float32),
    )

    def body(_, m2d):
        z1p = _sc_spmm(xW1_2, src2d, dst2d, m2d)
        z1, h1w2 = tc_fwd1(z1p, b1r, W2)
        z2p = _sc_spmm(h1w2.reshape(2 * n, H), src2d, dst2d, m2d)
        dz2, corr = tc_dz2(z2p, b2r, v2, m2d, t_arr)
        ge2, sp = _sc_bwd1(dz2.reshape(2 * n, H), h1w2.reshape(2 * n, H),
                           src2d, dst2d, m2d)
        dz1 = tc_dz1(sp, W2, z1)
        return _sc_bwd2(dz1, xW1, src2d, dst2d, m2d, ge2, corr[0, :16])

    m2d = lax.fori_loop(0, steps, body, m2d)
    return m2d.reshape(E_PAD)[:E]


# speedup vs baseline: 1.1186x; 1.1186x over previous
import functools

import jax
import jax.numpy as jnp
from jax import lax
from jax.experimental import pallas as pl
from jax.experimental.pallas import tpu as pltpu
from jax.experimental.pallas import tpu_sc as plsc

N = 10000
D = 128
H = D // 2
E = 160000
NC = 2
NS = 16
NW = NC * NS
C = 128
RPW = 40
E_PAD = NW * RPW * C
ROWS = E_PAD // C
REAL_ROWS = E // C
NPT = 624
ZR = 104
REM = N - NS * NPT

_MESH = plsc.VectorSubcoreMesh(
    core_axis_name="c", subcore_axis_name="s", num_cores=NC, num_subcores=NS)

_SC_PARAMS = pltpu.CompilerParams(use_tc_tiling_on_sc=False)

_GDN = lax.GatherDimensionNumbers(
    offset_dims=(), collapsed_slice_dims=(0,), start_index_map=(0,))


def _tree_sum(ps):
    while len(ps) > 1:
        ps = [a + b for a, b in zip(ps[::2], ps[1::2])]
    return ps[0]


def _shuf(v, idx):
    return lax.gather(v, idx.reshape(16, 1), _GDN, (1,),
                      indices_are_sorted=False, unique_indices=False,
                      mode=lax.GatherScatterMode.PROMISE_IN_BOUNDS)


def _lane_bcast(v, i):
    return _shuf(v, jnp.full((16,), i, jnp.int32))


def _transpose_reduce(accs):
    lanes = lax.iota(jnp.int32, 16)
    for sh in (1, 2, 4, 8):
        bit = (lanes & sh) == 0
        xidx = lanes ^ sh
        nxt = []
        for k in range(0, len(accs), 2):
            a, b = accs[k], accs[k + 1]
            m1 = jnp.where(bit, a, b)
            m2 = jnp.where(bit, b, a)
            nxt.append(m1 + _shuf(m2, xidx))
        accs = nxt
    return accs[0]


def _worker_prelude(src_h, dst_h, m_h, idxs_v, idxd_v, m_v):
    c = lax.axis_index("c")
    s = lax.axis_index("s")
    w = s * NC + c
    r0 = w * RPW
    pltpu.sync_copy(src_h.at[pl.ds(r0, RPW)], idxs_v)
    pltpu.sync_copy(dst_h.at[pl.ds(r0, RPW)], idxd_v)
    pltpu.sync_copy(m_h.at[pl.ds(r0, RPW)], m_v)
    return c, s, r0


def _affine_idx(out_v, src_v, mul, add):
    def row(j, _):
        for q in range(C // 16):
            sl = pl.ds(16 * q, 16)
            out_v[j, sl] = src_v[j, sl] * mul + add
        return 0
    lax.fori_loop(0, RPW, row, 0)


def _zero_acc(rows_v, acc_sh, s):
    def zrow(i, _):
        for k in range(H // 16):
            rows_v[i, pl.ds(16 * k, 16)] = jnp.zeros((16,), jnp.float32)
        return 0
    lax.fori_loop(0, ZR, zrow, 0)
    for k in range(NPT // ZR):
        base = pl.multiple_of(s * NPT + k * ZR, 8)
        pltpu.sync_copy(rows_v.at[pl.ds(0, ZR)], acc_sh.at[pl.ds(base, ZR)])

    @pl.when(s == NS - 1)
    def _():
        pltpu.sync_copy(rows_v.at[pl.ds(0, REM)],
                        acc_sh.at[pl.ds(NS * NPT, REM)])
    plsc.subcore_barrier()


def _acc_to_hbm(acc_sh, out_slice, rows_v, s):
    plsc.subcore_barrier()
    for k in range(NPT // ZR):
        base = pl.multiple_of(s * NPT + k * ZR, 8)
        pltpu.sync_copy(acc_sh.at[pl.ds(base, ZR)], rows_v.at[pl.ds(0, ZR)])
        pltpu.sync_copy(rows_v.at[pl.ds(0, ZR)], out_slice.at[pl.ds(base, ZR)])

    @pl.when(s == NS - 1)
    def _():
        pltpu.sync_copy(acc_sh.at[pl.ds(NS * NPT, REM)],
                        rows_v.at[pl.ds(0, REM)])
        pltpu.sync_copy(rows_v.at[pl.ds(0, REM)],
                        out_slice.at[pl.ds(NS * NPT, REM)])
    plsc.subcore_barrier()


NBUF = 4
NBUF1 = 2


@functools.partial(
    pl.kernel, mesh=_MESH, compiler_params=_SC_PARAMS,
    out_type=jax.ShapeDtypeStruct((NC, 2, N, H), jnp.float32),
    scratch_types=[
        pltpu.VMEM((RPW, C), jnp.int32),
        pltpu.VMEM((RPW, C), jnp.int32),
        pltpu.VMEM((RPW, C), jnp.float32),
        [pltpu.VMEM((C, H), jnp.float32)] * NBUF,
        pltpu.VMEM_SHARED((N, H), jnp.float32),
        [pltpu.SemaphoreType.DMA] * NBUF,
        [pltpu.SemaphoreType.DMA] * NBUF,
    ],
)
def _sc_spmm(tab2_h, src_h, dst_h, m_h, out_h,
             idxs_v, idxd_v, m_v, rows_bufs, acc_sh, gsems, ssems):
    c, s, _ = _worker_prelude(src_h, dst_h, m_h, idxs_v, idxd_v, m_v)
    for h in range(2):
        _affine_idx(idxs_v, idxs_v, 2 if h == 0 else 1, h)
        _zero_acc(rows_bufs[0], acc_sh, s)

        def quad(jj, _):
            j0 = jj * NBUF
            gets = [pltpu.async_copy(tab2_h.at[idxs_v.at[j0 + k]],
                                     rows_bufs[k], gsems[k])
                    for k in range(NBUF)]
            puts = []
            for k in range(NBUF):
                gets[k].wait()
                j = j0 + k
                rows_v = rows_bufs[k]

                @plsc.parallel_loop(0, C // 16, unroll=2)
                def group(g, j=j, rows_v=rows_v):
                    mv = m_v[j, pl.ds(16 * g, 16)]
                    for i in range(16):
                        e = 16 * g + i
                        mvec = _lane_bcast(mv, i)
                        for q in range(H // 16):
                            sl = pl.ds(16 * q, 16)
                            rows_v[e, sl] = rows_v[e, sl] * mvec
                puts.append(pltpu.async_copy(
                    rows_v, acc_sh.at[idxd_v.at[j]], ssems[k], add=True))
            for p in puts:
                p.wait()
            return 0
        lax.fori_loop(0, RPW // NBUF, quad, 0)
        _acc_to_hbm(acc_sh, out_h.at[c, h], rows_bufs[0], s)


@functools.partial(
    pl.kernel, mesh=_MESH, compiler_params=_SC_PARAMS,
    out_type=(jax.ShapeDtypeStruct((ROWS, C), jnp.float32),
              jax.ShapeDtypeStruct((NC, 2, N, H), jnp.float32)),
    scratch_types=[
        pltpu.VMEM((RPW, C), jnp.int32),
        pltpu.VMEM((RPW, C), jnp.int32),
        pltpu.VMEM((RPW, C), jnp.float32),
        pltpu.VMEM((RPW, C), jnp.float32),
        pltpu.VMEM((RPW, C), jnp.int32),
        [pltpu.VMEM((C, H), jnp.float32)] * NBUF1,
        [pltpu.VMEM((C, H), jnp.float32)] * NBUF1,
        pltpu.VMEM_SHARED((N, H), jnp.float32),
        [pltpu.SemaphoreType.DMA] * NBUF1,
        [pltpu.SemaphoreType.DMA] * NBUF1,
        [pltpu.SemaphoreType.DMA] * NBUF1,
    ],
)
def _sc_bwd1(dz2_h, h1w2_h, src_h, dst_h, m_h, ge2_h, out_h,
             idxs_v, ga_v, m_v, ge2_v, gb_v, abufs, bbufs,
             acc_sh, asems, bsems, ssems):
    c, s, r0 = _worker_prelude(src_h, dst_h, m_h, idxs_v, ga_v, m_v)
    for h in range(2):
        _affine_idx(ga_v, ga_v, 2 if h == 0 else 1, h)
        if h == 0:
            _affine_idx(gb_v, idxs_v, 2, 0)
        else:
            _affine_idx(gb_v, gb_v, 1, 1)
        _zero_acc(abufs[0], acc_sh, s)

        def quad(jj, _):
            j0 = jj * NBUF1
            gets = [(pltpu.async_copy(dz2_h.at[ga_v.at[j0 + k]],
                                      abufs[k], asems[k]),
                     pltpu.async_copy(h1w2_h.at[gb_v.at[j0 + k]],
                                      bbufs[k], bsems[k]))
                    for k in range(NBUF1)]
            puts = []
            for k in range(NBUF1):
                gets[k][0].wait()
                gets[k][1].wait()
                j = j0 + k
                rowsa_v, rowsb_v = abufs[k], bbufs[k]

                @plsc.parallel_loop(0, C // 16, unroll=2)
                def group(g, j=j, rowsa_v=rowsa_v, rowsb_v=rowsb_v):
                    sl16 = pl.ds(16 * g, 16)
                    mv = m_v[j, sl16]
                    accs = []
                    for i in range(16):
                        e = 16 * g + i
                        mvec = _lane_bcast(mv, i)
                        ps = []
                        for q in range(H // 16):
                            sl = pl.ds(16 * q, 16)
                            a = rowsa_v[e, sl]
                            ps.append(a * rowsb_v[e, sl])
                            rowsa_v[e, sl] = a * mvec
                        accs.append(_tree_sum(ps))
                    gvec = _transpose_reduce(accs)
                    if h == 0:
                        ge2_v[j, sl16] = gvec
                    else:
                        ge2_v[j, sl16] = ge2_v[j, sl16] + gvec
                puts.append(pltpu.async_copy(
                    rowsa_v, acc_sh.at[idxs_v.at[j]], ssems[k], add=True))
            for p in puts:
                p.wait()
            return 0
        lax.fori_loop(0, RPW // NBUF1, quad, 0)
        _acc_to_hbm(acc_sh, out_h.at[c, h], abufs[0], s)
    pltpu.sync_copy(ge2_v, ge2_h.at[pl.ds(r0, RPW)])


@functools.partial(
    pl.kernel, mesh=_MESH, compiler_params=_SC_PARAMS,
    out_type=jax.ShapeDtypeStruct((ROWS, C), jnp.float32),
    scratch_types=[
        pltpu.VMEM((RPW, C), jnp.int32),
        pltpu.VMEM((RPW, C), jnp.int32),
        pltpu.VMEM((RPW, C), jnp.float32),
        pltpu.VMEM((RPW, C), jnp.float32),
        pltpu.VMEM((16,), jnp.float32),
        [pltpu.VMEM((C, D), jnp.float32)] * 2,
        [pltpu.VMEM((C, D), jnp.float32)] * 2,
        [pltpu.SemaphoreType.DMA] * 2,
        [pltpu.SemaphoreType.DMA] * 2,
    ],
)
def _sc_bwd2(dz1_h, xw1_h, src_h, dst_h, m_h, ge2_h, corr_h, mout_h,
             idxs_v, idxd_v, m_v, ge2_v, corr_v, abufs, bbufs,
             asems, bsems):
    _, _, r0 = _worker_prelude(src_h, dst_h, m_h, idxs_v, idxd_v, m_v)
    pltpu.sync_copy(ge2_h.at[pl.ds(r0, RPW)], ge2_v)
    pltpu.sync_copy(corr_h, corr_v)
    corr = corr_v[...]

    def pair(jj, _):
        j0 = jj * 2
        gets = [(pltpu.async_copy(dz1_h.at[idxd_v.at[j0 + k]],
                                  abufs[k], asems[k]),
                 pltpu.async_copy(xw1_h.at[idxs_v.at[j0 + k]],
                                  bbufs[k], bsems[k]))
                for k in range(2)]
        for k in range(2):
            gets[k][0].wait()
            gets[k][1].wait()
            j = j0 + k
            rowsa_v, rowsb_v = abufs[k], bbufs[k]

            gj = r0 + j
            live = jnp.where(gj < REAL_ROWS, 1.0, 0.0).astype(jnp.float32)

            @plsc.parallel_loop(0, C // 16, unroll=2)
            def group(g, j=j, rowsa_v=rowsa_v, rowsb_v=rowsb_v, live=live):
                sl16 = pl.ds(16 * g, 16)
                accs = []
                for i in range(16):
                    e = 16 * g + i
                    ps = [rowsa_v[e, pl.ds(16 * q, 16)]
                          * rowsb_v[e, pl.ds(16 * q, 16)]
                          for q in range(D // 16)]
                    accs.append(_tree_sum(ps))
                gtot = _transpose_reduce(accs) + ge2_v[j, sl16] + corr
                m_v[j, sl16] = jnp.clip(m_v[j, sl16] + 0.005 * gtot,
                                        0.0, 1.0) * live
        return 0
    lax.fori_loop(0, RPW // 2, pair, 0)
    pltpu.sync_copy(m_v, mout_h.at[pl.ds(r0, RPW)])


def _tc_pre_body(x_ref, w1_ref, o_ref):
    o_ref[...] = jnp.dot(x_ref[...], w1_ref[...],
                         preferred_element_type=jnp.float32)


def _assemble(p_ref, b_ref):
    z = jnp.concatenate([p_ref[0, 0] + p_ref[1, 0],
                         p_ref[0, 1] + p_ref[1, 1]], axis=-1)
    return z + b_ref[...]


def _tc_fwd1_body(z1p_ref, b1_ref, w2_ref, z1_ref, h1w2_ref):
    z1 = _assemble(z1p_ref, b1_ref)
    z1_ref[...] = z1
    h1 = jnp.maximum(z1, 0.0)
    h1w2_ref[...] = jnp.dot(h1, w2_ref[...],
                            preferred_element_type=jnp.float32)


def _tc_dz2_body(z2p_ref, b2_ref, v2_ref, m_ref, t_ref, dz2_ref, corr_ref):
    z2 = _assemble(z2p_ref, b2_ref)
    dz2_ref[...] = jnp.where(z2 > 0.0, v2_ref[...], 0.0)
    corr = 2.0 * (t_ref[0, 0] - jnp.sum(m_ref[...]))
    corr_ref[...] = jnp.full((1, 128), corr, jnp.float32)


def _tc_dz1_body(sp_ref, w2_ref, z1_ref, dz1_ref):
    srow = jnp.concatenate([sp_ref[0, 0] + sp_ref[1, 0],
                            sp_ref[0, 1] + sp_ref[1, 1]], axis=-1)
    dh1 = lax.dot_general(srow, w2_ref[...], (((1,), (1,)), ((), ())),
                          preferred_element_type=jnp.float32)
    dz1_ref[...] = jnp.where(z1_ref[...] > 0.0, dh1, 0.0)


def kernel(x, edge_index, nodes, target_label, steps, W1, b1, W2, b2, Wlin, blin):
    n = x.shape[0]
    src = edge_index[0].astype(jnp.int32)
    dst = edge_index[1].astype(jnp.int32)
    pad = E_PAD - E
    src2d = jnp.concatenate([src, jnp.zeros((pad,), jnp.int32)]).reshape(ROWS, C)
    dst2d = jnp.concatenate([dst, jnp.zeros((pad,), jnp.int32)]).reshape(ROWS, C)
    m2d = jnp.concatenate([jnp.full((E,), 0.5, jnp.float32),
                           jnp.zeros((pad,), jnp.float32)]).reshape(ROWS, C)
    v2 = (Wlin[:, target_label] / nodes).astype(jnp.float32).reshape(1, D)
    t_arr = jnp.where(target_label == 0, 9.0, 8.0).astype(jnp.float32).reshape(1, 1)
    b1r = b1.reshape(1, D)
    b2r = b2.reshape(1, D)

    xW1 = pl.pallas_call(
        _tc_pre_body,
        out_shape=jax.ShapeDtypeStruct((n, D), jnp.float32),
    )(x, W1)
    xW1_2 = xW1.reshape(2 * n, H)

    tc_fwd1 = pl.pallas_call(
        _tc_fwd1_body,
        out_shape=(jax.ShapeDtypeStruct((n, D), jnp.float32),
                   jax.ShapeDtypeStruct((n, D), jnp.float32)),
    )
    tc_dz2 = pl.pallas_call(
        _tc_dz2_body,
        out_shape=(jax.ShapeDtypeStruct((n, D), jnp.float32),
                   jax.ShapeDtypeStruct((1, 128), jnp.float32)),
    )
    tc_dz1 = pl.pallas_call(
        _tc_dz1_body,
        out_shape=jax.ShapeDtypeStruct((n, D), jnp.float32),
    )

    def body(_, m2d):
        z1p = _sc_spmm(xW1_2, src2d, dst2d, m2d)
        z1, h1w2 = tc_fwd1(z1p, b1r, W2)
        z2p = _sc_spmm(h1w2.reshape(2 * n, H), src2d, dst2d, m2d)
        dz2, corr = tc_dz2(z2p, b2r, v2, m2d, t_arr)
        ge2, sp = _sc_bwd1(dz2.reshape(2 * n, H), h1w2.reshape(2 * n, H),
                           src2d, dst2d, m2d)
        dz1 = tc_dz1(sp, W2, z1)
        return _sc_bwd2(dz1, xW1, src2d, dst2d, m2d, ge2, corr[0, :16])

    m2d = lax.fori_loop(0, steps, body, m2d)
    return m2d.reshape(E_PAD)[:E]
